# baseline (speedup 1.0000x reference)
import functools

import jax
import jax.numpy as jnp
from jax import lax
from jax.experimental import pallas as pl
from jax.experimental.pallas import tpu as pltpu
from jax.experimental.pallas import tpu_sc as plsc

N, D, H, E, K = 2048, 768, 768, 8, 2
P = N * K
BLK = 128
S = P + E * BLK
G = S // BLK
GP = 48

_NC, _NS = 2, 16
NW = _NC * _NS


def _gate_body(x_ref, wg_ref, topi_ref, gp_ref):
    x = x_ref[...]
    logits = jnp.dot(x, wg_ref[...], preferred_element_type=jnp.float32)
    cols = jax.lax.broadcasted_iota(jnp.int32, logits.shape, 1)
    m1 = jnp.max(logits, axis=1, keepdims=True)
    a1 = jnp.argmax(logits, axis=1).reshape(-1, 1)
    neg = jnp.full_like(logits, -jnp.inf)
    masked = jnp.where(cols == a1, neg, logits)
    m2 = jnp.max(masked, axis=1, keepdims=True)
    a2 = jnp.argmax(masked, axis=1).reshape(-1, 1)
    t = jnp.exp(m2 - m1)
    w1g = 1.0 / (1.0 + t)
    w2g = t / (1.0 + t)
    topi_ref[...] = jnp.concatenate([a1, a2], axis=1).astype(jnp.int32)
    gp_ref[...] = jnp.concatenate([w1g, w2g], axis=1)


def _gating(x, Wg):
    return pl.pallas_call(
        _gate_body,
        out_shape=(
            jax.ShapeDtypeStruct((N, K), jnp.int32),
            jax.ShapeDtypeStruct((N, K), jnp.float32),
        ),
    )(x, Wg)


_CHUNK = P // _NS


def _count_body(ef_hbm, cnt_hbm, e_v, cnt_v):
    cid = lax.axis_index("c")
    sid = lax.axis_index("s")
    lane = lax.iota(jnp.int32, 16)

    @pl.when(cid == 0)
    def _work():
        base = sid * _CHUNK
        pltpu.sync_copy(ef_hbm.at[pl.ds(base, _CHUNK)], e_v)
        counts = jnp.zeros((16,), jnp.int32)
        for i in range(_CHUNK // 16):
            v = e_v[pl.ds(i * 16, 16)]
            for e in range(E):
                c = jnp.sum((v == e).astype(jnp.int32))
                counts = jnp.where(lane == e, counts + c, counts)
        cnt_v[...] = counts
        pltpu.sync_copy(cnt_v, cnt_hbm.at[sid])


def _count(ef):
    mesh = plsc.VectorSubcoreMesh(core_axis_name="c", subcore_axis_name="s")
    return pl.kernel(
        _count_body,
        mesh=mesh,
        out_type=jax.ShapeDtypeStruct((_NS, 16), jnp.int32),
        scratch_types=[
            pltpu.VMEM((_CHUNK,), jnp.int32),
            pltpu.VMEM((16,), jnp.int32),
        ],
        compiler_params=pltpu.CompilerParams(needs_layout_passes=False),
    )(ef)


def _route_body(ef_hbm, gf_hbm, cnt_hbm, tok_hbm, gate_hbm, pos_hbm, blk_hbm,
                e_v, g2_v, tok_v, pos_v, idx2_v, allcnt_v, blk_v, sem):
    cid = lax.axis_index("c")
    sid = lax.axis_index("s")
    lane = lax.iota(jnp.int32, 16)

    @pl.when(cid == 0)
    def _work2():
        base = sid * _CHUNK
        pltpu.sync_copy(ef_hbm.at[pl.ds(base, _CHUNK)], e_v)
        for c in range(2):
            pltpu.sync_copy(gf_hbm.at[pl.ds(base + c * 128, 128)], g2_v.at[c])
        pltpu.sync_copy(cnt_hbm, allcnt_v)
        total = jnp.zeros((16,), jnp.int32)
        prefix = jnp.zeros((16,), jnp.int32)
        for t in range(_NS):
            row = allcnt_v[t]
            total = total + row
            prefix = prefix + jnp.where(t < sid, row, 0)
        aligned = (total + (BLK - 1)) & ~(BLK - 1)
        excl = plsc.cumsum(aligned) - aligned
        base_vec = excl + prefix

        b_s = [jnp.sum(jnp.where(lane == e, base_vec, 0)) for e in range(E)]

        runs = [jnp.int32(0) for _ in range(E)]
        for i in range(_CHUNK // 16):
            v = e_v[pl.ds(i * 16, 16)]
            posv = jnp.zeros((16,), jnp.int32)
            for e in range(E):
                m = v == e
                csum = plsc.cumsum(m.astype(jnp.int32))
                rank = csum - 1
                posv = jnp.where(m, rank + (runs[e] + b_s[e]), posv)
                runs[e] = runs[e] + jnp.sum(m.astype(jnp.int32))
            pos_v[pl.ds(i * 16, 16)] = posv
            idx2_v[i // 8, pl.ds((i % 8) * 16, 16)] = posv
            tok_v[pl.ds(i * 16, 16)] = (base + i * 16 + lane) & (N - 1)

        pltpu.sync_copy(pos_v, pos_hbm.at[pl.ds(base, _CHUNK)])
        for c in range(2):
            pltpu.async_copy(tok_v.at[pl.ds(c * 128, 128)],
                             tok_hbm.at[idx2_v.at[c]], sem).wait()
            pltpu.async_copy(g2_v.at[c],
                             gate_hbm.at[idx2_v.at[c]], sem).wait()

        @pl.when(sid == 0)
        def _blocks():
            e_starts = [jnp.sum(jnp.where(lane == e, excl, 0)) for e in range(E)]
            for c in range(GP // 16):
                g = (c * 16 + lane) * BLK
                be = jnp.zeros((16,), jnp.int32)
                for e in range(1, E):
                    be = be + (g >= e_starts[e]).astype(jnp.int32)
                blk_v[pl.ds(c * 16, 16)] = be
            pltpu.sync_copy(blk_v, blk_hbm)


def _route(ef, gf):
    cnt = _count(ef)
    mesh = plsc.VectorSubcoreMesh(core_axis_name="c", subcore_axis_name="s")
    return pl.kernel(
        _route_body,
        mesh=mesh,
        out_type=(
            jax.ShapeDtypeStruct((S,), jnp.int32),
            jax.ShapeDtypeStruct((S,), jnp.float32),
            jax.ShapeDtypeStruct((P,), jnp.int32),
            jax.ShapeDtypeStruct((GP,), jnp.int32),
        ),
        scratch_types=[
            pltpu.VMEM((_CHUNK,), jnp.int32),
            pltpu.VMEM((2, 128), jnp.float32),
            pltpu.VMEM((_CHUNK,), jnp.int32),
            pltpu.VMEM((_CHUNK,), jnp.int32),
            pltpu.VMEM((2, 128), jnp.int32),
            pltpu.VMEM((_NS, 16), jnp.int32),
            pltpu.VMEM((GP,), jnp.int32),
            pltpu.SemaphoreType.DMA,
        ],
        compiler_params=pltpu.CompilerParams(needs_layout_passes=False),
    )(ef, gf, cnt)


_RPW = S // NW
_GC = 80


def _gather_body(x_hbm, tok_hbm, xs_hbm, idx_v, buf_v, sem):
    wid = lax.axis_index("s") * _NC + lax.axis_index("c")
    base = wid * _RPW
    pltpu.sync_copy(tok_hbm.at[pl.ds(base, _RPW)], idx_v)
    for i in range(_RPW // 16):
        v = idx_v[pl.ds(i * 16, 16)]
        idx_v[pl.ds(i * 16, 16)] = jnp.minimum(jnp.maximum(v, 0), N - 1)
    for c in range(_RPW // _GC):
        pltpu.async_copy(x_hbm.at[idx_v.at[pl.ds(c * _GC, _GC)]], buf_v,
                         sem).wait()
        pltpu.sync_copy(buf_v, xs_hbm.at[pl.ds(base + c * _GC, _GC)])


def _gather_rows(x, srt_tok):
    mesh = plsc.VectorSubcoreMesh(core_axis_name="c", subcore_axis_name="s")
    return pl.kernel(
        _gather_body,
        mesh=mesh,
        out_type=jax.ShapeDtypeStruct((S, D), jnp.float32),
        scratch_types=[
            pltpu.VMEM((_RPW,), jnp.int32),
            pltpu.VMEM((_GC, D), jnp.float32),
            pltpu.SemaphoreType.DMA,
        ],
    )(x, srt_tok)


def _mlp_body(be_ref, xs_ref, w1_ref, b1_ref, w2_ref, b2_ref, gate_ref,
              out_ref):
    xb = xs_ref[...].astype(jnp.bfloat16)
    h = jnp.maximum(
        jnp.dot(xb, w1_ref[0].astype(jnp.bfloat16),
                preferred_element_type=jnp.float32) + b1_ref[0], 0.0)
    o = jnp.dot(h.astype(jnp.bfloat16), w2_ref[0].astype(jnp.bfloat16),
                preferred_element_type=jnp.float32) + b2_ref[0]
    out_ref[...] = o * gate_ref[...]


def _grouped_mlp(blk, xs, W1, b1, W2, b2, srt_gate):
    grid_spec = pltpu.PrefetchScalarGridSpec(
        num_scalar_prefetch=1,
        grid=(G,),
        in_specs=[
            pl.BlockSpec((BLK, D), lambda g, be: (g, 0)),
            pl.BlockSpec((1, D, H), lambda g, be: (be[g], 0, 0)),
            pl.BlockSpec((1, 1, H), lambda g, be: (be[g], 0, 0)),
            pl.BlockSpec((1, H, D), lambda g, be: (be[g], 0, 0)),
            pl.BlockSpec((1, 1, D), lambda g, be: (be[g], 0, 0)),
            pl.BlockSpec((BLK, 1), lambda g, be: (g, 0)),
        ],
        out_specs=pl.BlockSpec((BLK, D), lambda g, be: (g, 0)),
    )
    return pl.pallas_call(
        _mlp_body,
        grid_spec=grid_spec,
        out_shape=jax.ShapeDtypeStruct((S, D), jnp.float32),
    )(blk, xs, W1, b1.reshape(E, 1, H), W2, b2.reshape(E, 1, D),
      srt_gate.reshape(S, 1))


_TPW = N // NW


def _combine_body(out_hbm, pos_hbm, y_hbm, idx0_v, idx1_v, bufa_v, bufb_v,
                  sem):
    wid = lax.axis_index("s") * _NC + lax.axis_index("c")
    base = wid * _TPW
    pltpu.sync_copy(pos_hbm.at[pl.ds(base, _TPW)], idx0_v)
    pltpu.sync_copy(pos_hbm.at[pl.ds(N + base, _TPW)], idx1_v)
    pltpu.async_copy(out_hbm.at[idx0_v], bufa_v, sem).wait()
    pltpu.async_copy(out_hbm.at[idx1_v], bufb_v, sem).wait()

    def row(r, _):
        for k in range(D // 16):
            sl = pl.ds(k * 16, 16)
            bufa_v[r, sl] = bufa_v[r, sl] + bufb_v[r, sl]
        return _

    lax.fori_loop(0, _TPW, row, 0)
    pltpu.sync_copy(bufa_v, y_hbm.at[pl.ds(base, _TPW)])


def _combine(outs, pos):
    mesh = plsc.VectorSubcoreMesh(core_axis_name="c", subcore_axis_name="s")
    return pl.kernel(
        _combine_body,
        mesh=mesh,
        out_type=jax.ShapeDtypeStruct((N, D), jnp.float32),
        scratch_types=[
            pltpu.VMEM((_TPW,), jnp.int32),
            pltpu.VMEM((_TPW,), jnp.int32),
            pltpu.VMEM((_TPW, D), jnp.float32),
            pltpu.VMEM((_TPW, D), jnp.float32),
            pltpu.SemaphoreType.DMA,
        ],
    )(outs, pos)


def kernel(x, Wg, W1, b1, W2, b2):
    topi, gp = _gating(x, Wg)
    ef = topi.T.reshape(-1)
    gf = gp.T.reshape(-1)
    srt_tok, srt_gate, pos, blk = _route(ef, gf)
    xs = _gather_rows(x, srt_tok)
    outs = _grouped_mlp(blk, xs, W1, b1, W2, b2, srt_gate)
    return _combine(outs, pos)


# speedup vs baseline: 1.0536x; 1.0536x over previous
import jax
import jax.numpy as jnp
from jax import lax
from jax.experimental import pallas as pl
from jax.experimental.pallas import tpu as pltpu
from jax.experimental.pallas import tpu_sc as plsc

N, D, H, E, K = 2048, 768, 768, 8, 2
P = N * K
BLK = 128
S = P + E * BLK
G = S // BLK
GP = 48
_CH = 256

_NC, _NS = 2, 16
NW = _NC * _NS


def _gate_body(x_ref, wg_ref, gp_ref, pos_ref, blk_ref):
    x = x_ref[...]
    logits = jnp.dot(x, wg_ref[...], preferred_element_type=jnp.float32)
    cols = jax.lax.broadcasted_iota(jnp.int32, logits.shape, 1)
    m1 = jnp.max(logits, axis=1, keepdims=True)
    a1 = jnp.argmax(logits, axis=1).reshape(-1, 1)
    neg = jnp.full_like(logits, -jnp.inf)
    masked = jnp.where(cols == a1, neg, logits)
    m2 = jnp.max(masked, axis=1, keepdims=True)
    a2 = jnp.argmax(masked, axis=1).reshape(-1, 1)
    t = jnp.exp(m2 - m1)
    w1g = 1.0 / (1.0 + t)
    w2g = t / (1.0 + t)
    gp_ref[...] = jnp.concatenate([w1g, w2g], axis=1)

    oh = jnp.concatenate([cols == a1, cols == a2], axis=0)
    ohf = oh.astype(jnp.float32)
    tot = jnp.sum(ohf, axis=0, keepdims=True).astype(jnp.int32)
    aligned = (tot + (BLK - 1)) & (-BLK)
    lane8 = jax.lax.broadcasted_iota(jnp.int32, (1, E), 1)
    seg_sc = []
    run = jnp.int32(0)
    for e in range(E):
        seg_sc.append(run)
        run = run + jnp.sum(jnp.where(lane8 == e, aligned, 0))
    seg_row = jnp.zeros((1, E), jnp.int32)
    for e in range(1, E):
        seg_row = seg_row + jnp.where(lane8 == e, seg_sc[e], 0)

    ri = jax.lax.broadcasted_iota(jnp.int32, (_CH, _CH), 0)
    ci = jax.lax.broadcasted_iota(jnp.int32, (_CH, _CH), 1)
    tril = (ri > ci).astype(jnp.float32)
    carry = jnp.zeros((1, E), jnp.float32)
    for c in range(P // _CH):
        ohc = ohf[c * _CH:(c + 1) * _CH]
        rk = jnp.dot(tril, ohc, preferred_element_type=jnp.float32) + carry
        carry = carry + jnp.sum(ohc, axis=0, keepdims=True)
        ohi = oh[c * _CH:(c + 1) * _CH]
        posc = jnp.sum(jnp.where(ohi, rk.astype(jnp.int32) + seg_row, 0),
                       axis=1, keepdims=True)
        pos_ref[pl.ds(c * _CH, _CH), :] = posc

    gidx = jax.lax.broadcasted_iota(jnp.int32, (1, GP), 1) * BLK
    be = jnp.zeros((1, GP), jnp.int32)
    for e in range(1, E):
        be = be + (gidx >= seg_sc[e]).astype(jnp.int32)
    blk_ref[...] = be


def _gating(x, Wg):
    return pl.pallas_call(
        _gate_body,
        out_shape=(
            jax.ShapeDtypeStruct((N, K), jnp.float32),
            jax.ShapeDtypeStruct((P, 1), jnp.int32),
            jax.ShapeDtypeStruct((1, GP), jnp.int32),
        ),
    )(x, Wg)


_SPW = P // NW


def _scatter_body(gf_hbm, pos_hbm, tok_hbm, gate_hbm, idx_v, g_v, tok_v,
                  sem_a, sem_b):
    wid = lax.axis_index("s") * _NC + lax.axis_index("c")
    lane = lax.iota(jnp.int32, 16)
    base = wid * _SPW
    pltpu.sync_copy(pos_hbm.at[pl.ds(base, _SPW)], idx_v)
    pltpu.sync_copy(gf_hbm.at[pl.ds(base, _SPW)], g_v)
    for i in range(_SPW // 16):
        tok_v[pl.ds(i * 16, 16)] = (base + i * 16 + lane) & (N - 1)
    ca = pltpu.async_copy(tok_v, tok_hbm.at[idx_v], sem_a)
    cb = pltpu.async_copy(g_v, gate_hbm.at[idx_v], sem_b)
    ca.wait()
    cb.wait()


def _scatter(gf, pos):
    mesh = plsc.VectorSubcoreMesh(core_axis_name="c", subcore_axis_name="s")
    return pl.kernel(
        _scatter_body,
        mesh=mesh,
        out_type=(
            jax.ShapeDtypeStruct((S,), jnp.int32),
            jax.ShapeDtypeStruct((S,), jnp.float32),
        ),
        scratch_types=[
            pltpu.VMEM((_SPW,), jnp.int32),
            pltpu.VMEM((_SPW,), jnp.float32),
            pltpu.VMEM((_SPW,), jnp.int32),
            pltpu.SemaphoreType.DMA,
            pltpu.SemaphoreType.DMA,
        ],
        compiler_params=pltpu.CompilerParams(needs_layout_passes=False),
    )(gf, pos)


_RPW = S // NW
_NCH = 4
_GC = _RPW // _NCH


def _gather_body(x_hbm, tok_hbm, xs_hbm, idx_v, b0, b1, b2,
                 g0, g1, g2, g3, s0, s1, s2, s3):
    wid = lax.axis_index("s") * _NC + lax.axis_index("c")
    base = wid * _RPW
    bufs = [b0, b1, b2]
    gsems = [g0, g1, g2, g3]
    ssems = [s0, s1, s2, s3]
    pltpu.sync_copy(tok_hbm.at[pl.ds(base, _RPW)], idx_v)
    for i in range(_RPW // 16):
        v = idx_v[pl.ds(i * 16, 16)]
        idx_v[pl.ds(i * 16, 16)] = jnp.minimum(jnp.maximum(v, 0), N - 1)

    def start_gather(c):
        return pltpu.async_copy(
            x_hbm.at[idx_v.at[pl.ds(c * _GC, _GC)]], bufs[c % 3], gsems[c])

    gh = [None] * _NCH
    sh = [None] * _NCH
    gh[0] = start_gather(0)
    gh[1] = start_gather(1)
    for c in range(_NCH):
        gh[c].wait()
        sh[c] = pltpu.async_copy(
            bufs[c % 3], xs_hbm.at[pl.ds(base + c * _GC, _GC)], ssems[c])
        if c + 2 < _NCH:
            if c >= 1:
                sh[c - 1].wait()
            gh[c + 2] = start_gather(c + 2)
    for c in range(_NCH - 3, _NCH):
        if c >= 0:
            sh[c].wait()


def _gather_rows(x, srt_tok):
    mesh = plsc.VectorSubcoreMesh(core_axis_name="c", subcore_axis_name="s")
    return pl.kernel(
        _gather_body,
        mesh=mesh,
        out_type=jax.ShapeDtypeStruct((S, D), jnp.float32),
        scratch_types=[
            pltpu.VMEM((_RPW,), jnp.int32),
            pltpu.VMEM((_GC, D), jnp.float32),
            pltpu.VMEM((_GC, D), jnp.float32),
            pltpu.VMEM((_GC, D), jnp.float32),
        ] + [pltpu.SemaphoreType.DMA] * 8,
        compiler_params=pltpu.CompilerParams(needs_layout_passes=False),
    )(x, srt_tok)


def _mlp_body(be_ref, xs_ref, w1_ref, b1_ref, w2_ref, b2_ref, gate_ref,
              out_ref):
    xb = xs_ref[...].astype(jnp.bfloat16)
    h = jnp.maximum(
        jnp.dot(xb, w1_ref[0].astype(jnp.bfloat16),
                preferred_element_type=jnp.float32) + b1_ref[0], 0.0)
    o = jnp.dot(h.astype(jnp.bfloat16), w2_ref[0].astype(jnp.bfloat16),
                preferred_element_type=jnp.float32) + b2_ref[0]
    out_ref[...] = o * gate_ref[...]


def _grouped_mlp(blk, xs, W1, b1, W2, b2, srt_gate):
    grid_spec = pltpu.PrefetchScalarGridSpec(
        num_scalar_prefetch=1,
        grid=(G,),
        in_specs=[
            pl.BlockSpec((BLK, D), lambda g, be: (g, 0)),
            pl.BlockSpec((1, D, H), lambda g, be: (be[g], 0, 0)),
            pl.BlockSpec((1, 1, H), lambda g, be: (be[g], 0, 0)),
            pl.BlockSpec((1, H, D), lambda g, be: (be[g], 0, 0)),
            pl.BlockSpec((1, 1, D), lambda g, be: (be[g], 0, 0)),
            pl.BlockSpec((BLK, 1), lambda g, be: (g, 0)),
        ],
        out_specs=pl.BlockSpec((BLK, D), lambda g, be: (g, 0)),
    )
    return pl.pallas_call(
        _mlp_body,
        grid_spec=grid_spec,
        out_shape=jax.ShapeDtypeStruct((S, D), jnp.float32),
    )(blk, xs, W1, b1.reshape(E, 1, H), W2, b2.reshape(E, 1, D),
      srt_gate.reshape(S, 1))


_TPW = N // NW


def _combine_body(out_hbm, pos_hbm, y_hbm, idx0_v, idx1_v, bufa_v, bufb_v,
                  sem_a, sem_b):
    wid = lax.axis_index("s") * _NC + lax.axis_index("c")
    base = wid * _TPW
    pltpu.sync_copy(pos_hbm.at[pl.ds(base, _TPW)], idx0_v)
    pltpu.sync_copy(pos_hbm.at[pl.ds(N + base, _TPW)], idx1_v)
    ca = pltpu.async_copy(out_hbm.at[idx0_v], bufa_v, sem_a)
    cb = pltpu.async_copy(out_hbm.at[idx1_v], bufb_v, sem_b)
    ca.wait()
    cb.wait()

    @plsc.parallel_loop(0, _TPW, 1, unroll=2)
    def _row(r):
        for k in range(D // 16):
            sl = pl.ds(k * 16, 16)
            bufa_v[r, sl] = bufa_v[r, sl] + bufb_v[r, sl]

    pltpu.sync_copy(bufa_v, y_hbm.at[pl.ds(base, _TPW)])


def _combine(outs, pos):
    mesh = plsc.VectorSubcoreMesh(core_axis_name="c", subcore_axis_name="s")
    return pl.kernel(
        _combine_body,
        mesh=mesh,
        out_type=jax.ShapeDtypeStruct((N, D), jnp.float32),
        scratch_types=[
            pltpu.VMEM((_TPW,), jnp.int32),
            pltpu.VMEM((_TPW,), jnp.int32),
            pltpu.VMEM((_TPW, D), jnp.float32),
            pltpu.VMEM((_TPW, D), jnp.float32),
            pltpu.SemaphoreType.DMA,
            pltpu.SemaphoreType.DMA,
        ],
        compiler_params=pltpu.CompilerParams(needs_layout_passes=False),
    )(outs, pos)


def kernel(x, Wg, W1, b1, W2, b2):
    gp, pos2, blk2 = _gating(x, Wg)
    gf = gp.T.reshape(-1)
    pos = pos2.reshape(-1)
    blk = blk2.reshape(-1)
    srt_tok, srt_gate = _scatter(gf, pos)
    xs = _gather_rows(x, srt_tok)
    outs = _grouped_mlp(blk, xs, W1, b1, W2, b2, srt_gate)
    return _combine(outs, pos)


# speedup vs baseline: 1.7790x; 1.6886x over previous
import jax
import jax.numpy as jnp
from jax import lax
from jax.experimental import pallas as pl
from jax.experimental.pallas import tpu as pltpu
from jax.experimental.pallas import tpu_sc as plsc

N, D, H, E, K = 2048, 768, 768, 8, 2
P = N * K
BLK = 128
S = P + E * BLK
G = S // BLK
GP = 48
_CH = 256

_NC, _NS = 2, 16
NW = _NC * _NS


def _gate_body(x_ref, wg_ref, gp_ref, pos_ref, blk_ref):
    x = x_ref[...]
    logits = jnp.dot(x, wg_ref[...], preferred_element_type=jnp.float32)
    cols = jax.lax.broadcasted_iota(jnp.int32, logits.shape, 1)
    m1 = jnp.max(logits, axis=1, keepdims=True)
    a1 = jnp.argmax(logits, axis=1).reshape(-1, 1)
    neg = jnp.full_like(logits, -jnp.inf)
    masked = jnp.where(cols == a1, neg, logits)
    m2 = jnp.max(masked, axis=1, keepdims=True)
    a2 = jnp.argmax(masked, axis=1).reshape(-1, 1)
    t = jnp.exp(m2 - m1)
    w1g = 1.0 / (1.0 + t)
    w2g = t / (1.0 + t)
    gp_ref[...] = jnp.concatenate([w1g, w2g], axis=1)

    oh = jnp.concatenate([cols == a1, cols == a2], axis=0)
    ohf = oh.astype(jnp.float32)
    tot = jnp.sum(ohf, axis=0, keepdims=True).astype(jnp.int32)
    aligned = (tot + (BLK - 1)) & (-BLK)
    lane8 = jax.lax.broadcasted_iota(jnp.int32, (1, E), 1)
    seg_sc = []
    run = jnp.int32(0)
    for e in range(E):
        seg_sc.append(run)
        run = run + jnp.sum(jnp.where(lane8 == e, aligned, 0))
    seg_row = jnp.zeros((1, E), jnp.int32)
    for e in range(1, E):
        seg_row = seg_row + jnp.where(lane8 == e, seg_sc[e], 0)

    ri = jax.lax.broadcasted_iota(jnp.int32, (_CH, _CH), 0)
    ci = jax.lax.broadcasted_iota(jnp.int32, (_CH, _CH), 1)
    tril = (ri > ci).astype(jnp.float32)
    carry = jnp.zeros((1, E), jnp.float32)
    for c in range(P // _CH):
        ohc = ohf[c * _CH:(c + 1) * _CH]
        rk = jnp.dot(tril, ohc, preferred_element_type=jnp.float32) + carry
        carry = carry + jnp.sum(ohc, axis=0, keepdims=True)
        ohi = oh[c * _CH:(c + 1) * _CH]
        posc = jnp.sum(jnp.where(ohi, rk.astype(jnp.int32) + seg_row, 0),
                       axis=1, keepdims=True)
        pos_ref[pl.ds(c * _CH, _CH), :] = posc

    gidx = jax.lax.broadcasted_iota(jnp.int32, (1, GP), 1) * BLK
    be = jnp.zeros((1, GP), jnp.int32)
    for e in range(1, E):
        be = be + (gidx >= seg_sc[e]).astype(jnp.int32)
    blk_ref[...] = be


def _gating(x, Wg):
    return pl.pallas_call(
        _gate_body,
        out_shape=(
            jax.ShapeDtypeStruct((N, K), jnp.float32),
            jax.ShapeDtypeStruct((P, 1), jnp.int32),
            jax.ShapeDtypeStruct((1, GP), jnp.int32),
        ),
    )(x, Wg)


_SPW = P // NW


def _dispatch_body(x_hbm, pos_hbm, xs_hbm, idx_v, buf_v, sem):
    wid = lax.axis_index("s") * _NC + lax.axis_index("c")
    base = pl.multiple_of(wid * _SPW, _SPW)
    tokbase = pl.multiple_of(base & (N - 1), _SPW)
    pltpu.sync_copy(pos_hbm.at[pl.ds(base, _SPW)], idx_v)
    pltpu.sync_copy(x_hbm.at[pl.ds(tokbase, _SPW)], buf_v)
    pltpu.async_copy(buf_v, xs_hbm.at[idx_v], sem).wait()


def _dispatch(x, pos):
    mesh = plsc.VectorSubcoreMesh(core_axis_name="c", subcore_axis_name="s")
    return pl.kernel(
        _dispatch_body,
        mesh=mesh,
        out_type=jax.ShapeDtypeStruct((S, D), jnp.float32),
        scratch_types=[
            pltpu.VMEM((_SPW,), jnp.int32),
            pltpu.VMEM((_SPW, D), jnp.float32),
            pltpu.SemaphoreType.DMA,
        ],
        compiler_params=pltpu.CompilerParams(needs_layout_passes=False),
    )(x, pos)


def _mlp_body(be_ref, xs_ref, w1_ref, b1_ref, w2_ref, b2_ref, out_ref):
    xb = xs_ref[...].astype(jnp.bfloat16)
    h = jnp.maximum(
        jnp.dot(xb, w1_ref[0].astype(jnp.bfloat16),
                preferred_element_type=jnp.float32) + b1_ref[0], 0.0)
    o = jnp.dot(h.astype(jnp.bfloat16), w2_ref[0].astype(jnp.bfloat16),
                preferred_element_type=jnp.float32) + b2_ref[0]
    out_ref[...] = o


def _grouped_mlp(blk, xs, W1, b1, W2, b2):
    grid_spec = pltpu.PrefetchScalarGridSpec(
        num_scalar_prefetch=1,
        grid=(G,),
        in_specs=[
            pl.BlockSpec((BLK, D), lambda g, be: (g, 0)),
            pl.BlockSpec((1, D, H), lambda g, be: (be[g], 0, 0)),
            pl.BlockSpec((1, 1, H), lambda g, be: (be[g], 0, 0)),
            pl.BlockSpec((1, H, D), lambda g, be: (be[g], 0, 0)),
            pl.BlockSpec((1, 1, D), lambda g, be: (be[g], 0, 0)),
        ],
        out_specs=pl.BlockSpec((BLK, D), lambda g, be: (g, 0)),
    )
    return pl.pallas_call(
        _mlp_body,
        grid_spec=grid_spec,
        out_shape=jax.ShapeDtypeStruct((S, D), jnp.float32),
    )(blk, xs, W1, b1.reshape(E, 1, H), W2, b2.reshape(E, 1, D))


_TPW = N // NW


def _combine_body(out_hbm, pos_hbm, gf_hbm, y_hbm, idx0_v, idx1_v, g0_v, g1_v,
                  bufa_v, bufb_v, sem_a, sem_b):
    wid = lax.axis_index("s") * _NC + lax.axis_index("c")
    base = pl.multiple_of(wid * _TPW, _TPW)
    lane = lax.iota(jnp.int32, 16)
    pltpu.sync_copy(pos_hbm.at[pl.ds(base, _TPW)], idx0_v)
    pltpu.sync_copy(pos_hbm.at[pl.ds(N + base, _TPW)], idx1_v)
    pltpu.sync_copy(gf_hbm.at[pl.ds(base, _TPW)], g0_v)
    pltpu.sync_copy(gf_hbm.at[pl.ds(N + base, _TPW)], g1_v)
    ca = pltpu.async_copy(out_hbm.at[idx0_v], bufa_v, sem_a)
    cb = pltpu.async_copy(out_hbm.at[idx1_v], bufb_v, sem_b)
    ca.wait()
    cb.wait()

    def grp(g4, _):
        gva = g0_v[pl.ds(g4 * 16, 16)]
        gvb = g1_v[pl.ds(g4 * 16, 16)]
        for r16 in range(16):
            ga = jnp.sum(jnp.where(lane == r16, gva, 0.0))
            gb = jnp.sum(jnp.where(lane == r16, gvb, 0.0))
            r = g4 * 16 + r16
            for k in range(D // 16):
                sl = pl.ds(k * 16, 16)
                bufa_v[r, sl] = bufa_v[r, sl] * ga + bufb_v[r, sl] * gb
        return _

    lax.fori_loop(0, _TPW // 16, grp, 0)
    pltpu.sync_copy(bufa_v, y_hbm.at[pl.ds(base, _TPW)])


def _combine(outs, pos, gf):
    mesh = plsc.VectorSubcoreMesh(core_axis_name="c", subcore_axis_name="s")
    return pl.kernel(
        _combine_body,
        mesh=mesh,
        out_type=jax.ShapeDtypeStruct((N, D), jnp.float32),
        scratch_types=[
            pltpu.VMEM((_TPW,), jnp.int32),
            pltpu.VMEM((_TPW,), jnp.int32),
            pltpu.VMEM((_TPW,), jnp.float32),
            pltpu.VMEM((_TPW,), jnp.float32),
            pltpu.VMEM((_TPW, D), jnp.float32),
            pltpu.VMEM((_TPW, D), jnp.float32),
            pltpu.SemaphoreType.DMA,
            pltpu.SemaphoreType.DMA,
        ],
        compiler_params=pltpu.CompilerParams(needs_layout_passes=False),
    )(outs, pos, gf)


def kernel(x, Wg, W1, b1, W2, b2):
    gp, pos2, blk2 = _gating(x, Wg)
    gf = gp.T.reshape(-1)
    pos = pos2.reshape(-1)
    blk = blk2.reshape(-1)
    xs = _dispatch(x, pos)
    outs = _grouped_mlp(blk, xs, W1, b1, W2, b2)
    return _combine(outs, pos, gf)


# speedup vs baseline: 1.9219x; 1.0803x over previous
import jax
import jax.numpy as jnp
from jax import lax
from jax.experimental import pallas as pl
from jax.experimental.pallas import tpu as pltpu
from jax.experimental.pallas import tpu_sc as plsc

N, D, H, E, K = 2048, 768, 768, 8, 2
P = N * K
BLK = 128
S = P + E * BLK
G = S // BLK
GP = 48
_CH = 256

_NC, _NS = 2, 16
NW = _NC * _NS


def _gate_body(x_ref, wg_ref, gp_ref, pos_ref, blk_ref):
    x = x_ref[...]
    logits = jnp.dot(x, wg_ref[...], preferred_element_type=jnp.float32)
    cols = jax.lax.broadcasted_iota(jnp.int32, logits.shape, 1)
    m1 = jnp.max(logits, axis=1, keepdims=True)
    a1 = jnp.argmax(logits, axis=1).reshape(-1, 1)
    neg = jnp.full_like(logits, -jnp.inf)
    masked = jnp.where(cols == a1, neg, logits)
    m2 = jnp.max(masked, axis=1, keepdims=True)
    a2 = jnp.argmax(masked, axis=1).reshape(-1, 1)
    t = jnp.exp(m2 - m1)
    w1g = 1.0 / (1.0 + t)
    w2g = t / (1.0 + t)
    gp_ref[...] = jnp.concatenate([w1g, w2g], axis=1)

    oh = jnp.concatenate([cols == a1, cols == a2], axis=0)
    ohf = oh.astype(jnp.float32)
    tot = jnp.sum(ohf, axis=0, keepdims=True).astype(jnp.int32)
    aligned = (tot + (BLK - 1)) & (-BLK)
    lane8 = jax.lax.broadcasted_iota(jnp.int32, (1, E), 1)
    seg_sc = []
    run = jnp.int32(0)
    for e in range(E):
        seg_sc.append(run)
        run = run + jnp.sum(jnp.where(lane8 == e, aligned, 0))
    seg_row = jnp.zeros((1, E), jnp.int32)
    for e in range(1, E):
        seg_row = seg_row + jnp.where(lane8 == e, seg_sc[e], 0)

    ri = jax.lax.broadcasted_iota(jnp.int32, (_CH, _CH), 0)
    ci = jax.lax.broadcasted_iota(jnp.int32, (_CH, _CH), 1)
    tril = (ri > ci).astype(jnp.float32)
    carry = jnp.zeros((1, E), jnp.float32)
    for c in range(P // _CH):
        ohc = ohf[c * _CH:(c + 1) * _CH]
        rk = jnp.dot(tril, ohc, preferred_element_type=jnp.float32) + carry
        carry = carry + jnp.sum(ohc, axis=0, keepdims=True)
        ohi = oh[c * _CH:(c + 1) * _CH]
        posc = jnp.sum(jnp.where(ohi, rk.astype(jnp.int32) + seg_row, 0),
                       axis=1, keepdims=True)
        pos_ref[pl.ds(c * _CH, _CH), :] = posc

    gidx = jax.lax.broadcasted_iota(jnp.int32, (1, GP), 1) * BLK
    be = jnp.zeros((1, GP), jnp.int32)
    for e in range(1, E):
        be = be + (gidx >= seg_sc[e]).astype(jnp.int32)
    blk_ref[...] = be


def _gating(x, Wg):
    return pl.pallas_call(
        _gate_body,
        out_shape=(
            jax.ShapeDtypeStruct((N, K), jnp.float32),
            jax.ShapeDtypeStruct((P, 1), jnp.int32),
            jax.ShapeDtypeStruct((1, GP), jnp.int32),
        ),
    )(x, Wg)


_SPW = P // NW


def _dispatch_body(x_hbm, pos_hbm, xs_hbm, idx_v, buf_v, sem):
    wid = lax.axis_index("s") * _NC + lax.axis_index("c")
    base = pl.multiple_of(wid * _SPW, _SPW)
    tokbase = pl.multiple_of(base & (N - 1), _SPW)
    pltpu.sync_copy(pos_hbm.at[pl.ds(base, _SPW)], idx_v)
    pltpu.sync_copy(x_hbm.at[pl.ds(tokbase, _SPW)], buf_v)
    pltpu.async_copy(buf_v, xs_hbm.at[idx_v], sem).wait()


def _dispatch(x, pos):
    mesh = plsc.VectorSubcoreMesh(core_axis_name="c", subcore_axis_name="s")
    return pl.kernel(
        _dispatch_body,
        mesh=mesh,
        out_type=jax.ShapeDtypeStruct((S, D), jnp.float32),
        scratch_types=[
            pltpu.VMEM((_SPW,), jnp.int32),
            pltpu.VMEM((_SPW, D), jnp.float32),
            pltpu.SemaphoreType.DMA,
        ],
        compiler_params=pltpu.CompilerParams(needs_layout_passes=False),
    )(x, pos)


def _mlp_body(be_ref, xs_ref, w1_ref, b1_ref, w2_ref, b2_ref, out_ref):
    xb = xs_ref[...].astype(jnp.bfloat16)
    h = jnp.maximum(
        jnp.dot(xb, w1_ref[0].astype(jnp.bfloat16),
                preferred_element_type=jnp.float32) + b1_ref[0], 0.0)
    o = jnp.dot(h.astype(jnp.bfloat16), w2_ref[0].astype(jnp.bfloat16),
                preferred_element_type=jnp.float32) + b2_ref[0]
    out_ref[...] = o


def _grouped_mlp(blk, xs, W1, b1, W2, b2):
    grid_spec = pltpu.PrefetchScalarGridSpec(
        num_scalar_prefetch=1,
        grid=(G,),
        in_specs=[
            pl.BlockSpec((BLK, D), lambda g, be: (g, 0)),
            pl.BlockSpec((1, D, H), lambda g, be: (be[g], 0, 0)),
            pl.BlockSpec((1, 1, H), lambda g, be: (be[g], 0, 0)),
            pl.BlockSpec((1, H, D), lambda g, be: (be[g], 0, 0)),
            pl.BlockSpec((1, 1, D), lambda g, be: (be[g], 0, 0)),
        ],
        out_specs=pl.BlockSpec((BLK, D), lambda g, be: (g, 0)),
    )
    return pl.pallas_call(
        _mlp_body,
        grid_spec=grid_spec,
        out_shape=jax.ShapeDtypeStruct((S, D), jnp.float32),
    )(blk, xs, W1, b1.reshape(E, 1, H), W2, b2.reshape(E, 1, D))


_TPW = N // NW


def _combine_body(out_hbm, pos_hbm, gf_hbm, y_hbm, idx0_v, idx1_v, g_v,
                  bufa_v, bufb_v, sem_a, sem_b):
    wid = lax.axis_index("s") * _NC + lax.axis_index("c")
    base = pl.multiple_of(wid * _TPW, _TPW)
    lane = lax.iota(jnp.int32, 16)
    pltpu.sync_copy(pos_hbm.at[pl.ds(base, _TPW)], idx0_v)
    pltpu.sync_copy(pos_hbm.at[pl.ds(N + base, _TPW)], idx1_v)
    pltpu.sync_copy(gf_hbm.at[pl.ds(2 * base, 2 * _TPW)], g_v)
    ca = pltpu.async_copy(out_hbm.at[idx0_v], bufa_v, sem_a)
    cb = pltpu.async_copy(out_hbm.at[idx1_v], bufb_v, sem_b)
    ca.wait()
    cb.wait()

    def grp(g8, _):
        gv = g_v[pl.ds(g8 * 16, 16)]
        for r8 in range(8):
            ga = jnp.sum(jnp.where(lane == 2 * r8, gv, 0.0))
            gb = jnp.sum(jnp.where(lane == 2 * r8 + 1, gv, 0.0))
            r = g8 * 8 + r8
            for k in range(D // 16):
                sl = pl.ds(k * 16, 16)
                bufa_v[r, sl] = bufa_v[r, sl] * ga + bufb_v[r, sl] * gb
        return _

    lax.fori_loop(0, _TPW // 8, grp, 0)
    pltpu.sync_copy(bufa_v, y_hbm.at[pl.ds(base, _TPW)])


def _combine(outs, pos, gf):
    mesh = plsc.VectorSubcoreMesh(core_axis_name="c", subcore_axis_name="s")
    return pl.kernel(
        _combine_body,
        mesh=mesh,
        out_type=jax.ShapeDtypeStruct((N, D), jnp.float32),
        scratch_types=[
            pltpu.VMEM((_TPW,), jnp.int32),
            pltpu.VMEM((_TPW,), jnp.int32),
            pltpu.VMEM((2 * _TPW,), jnp.float32),
            pltpu.VMEM((_TPW, D), jnp.float32),
            pltpu.VMEM((_TPW, D), jnp.float32),
            pltpu.SemaphoreType.DMA,
            pltpu.SemaphoreType.DMA,
        ],
        compiler_params=pltpu.CompilerParams(needs_layout_passes=False),
    )(outs, pos, gf)


def kernel(x, Wg, W1, b1, W2, b2):
    gp, pos2, blk2 = _gating(x, Wg)
    gf = gp.reshape(-1)
    pos = pos2.reshape(-1)
    blk = blk2.reshape(-1)
    xs = _dispatch(x, pos)
    outs = _grouped_mlp(blk, xs, W1, b1, W2, b2)
    return _combine(outs, pos, gf)


# speedup vs baseline: 2.0516x; 1.0675x over previous
import jax
import jax.numpy as jnp
from jax import lax
from jax.experimental import pallas as pl
from jax.experimental.pallas import tpu as pltpu
from jax.experimental.pallas import tpu_sc as plsc

N, D, H, E, K = 2048, 768, 768, 8, 2
P = N * K
BLK = 256
S = P + E * BLK
G = S // BLK
GP = 48
_CH = 256

_NC, _NS = 2, 16
NW = _NC * _NS


def _gate_body(x_ref, wg_ref, gp_ref, pos_ref, blk_ref):
    x = x_ref[...]
    logits = jnp.dot(x, wg_ref[...], preferred_element_type=jnp.float32)
    cols = jax.lax.broadcasted_iota(jnp.int32, logits.shape, 1)
    m1 = jnp.max(logits, axis=1, keepdims=True)
    a1 = jnp.argmax(logits, axis=1).reshape(-1, 1)
    neg = jnp.full_like(logits, -jnp.inf)
    masked = jnp.where(cols == a1, neg, logits)
    m2 = jnp.max(masked, axis=1, keepdims=True)
    a2 = jnp.argmax(masked, axis=1).reshape(-1, 1)
    t = jnp.exp(m2 - m1)
    w1g = 1.0 / (1.0 + t)
    w2g = t / (1.0 + t)
    gp_ref[...] = jnp.concatenate([w1g, w2g], axis=1)

    oh = jnp.concatenate([cols == a1, cols == a2], axis=0)
    ohf = oh.astype(jnp.float32)
    tot = jnp.sum(ohf, axis=0, keepdims=True).astype(jnp.int32)
    aligned = (tot + (BLK - 1)) & (-BLK)
    lane8 = jax.lax.broadcasted_iota(jnp.int32, (1, E), 1)
    seg_sc = []
    run = jnp.int32(0)
    for e in range(E):
        seg_sc.append(run)
        run = run + jnp.sum(jnp.where(lane8 == e, aligned, 0))
    seg_row = jnp.zeros((1, E), jnp.int32)
    for e in range(1, E):
        seg_row = seg_row + jnp.where(lane8 == e, seg_sc[e], 0)

    ri = jax.lax.broadcasted_iota(jnp.int32, (_CH, _CH), 0)
    ci = jax.lax.broadcasted_iota(jnp.int32, (_CH, _CH), 1)
    tril = (ri > ci).astype(jnp.float32)
    carry = jnp.zeros((1, E), jnp.float32)
    for c in range(P // _CH):
        ohc = ohf[c * _CH:(c + 1) * _CH]
        rk = jnp.dot(tril, ohc, preferred_element_type=jnp.float32) + carry
        carry = carry + jnp.sum(ohc, axis=0, keepdims=True)
        ohi = oh[c * _CH:(c + 1) * _CH]
        posc = jnp.sum(jnp.where(ohi, rk.astype(jnp.int32) + seg_row, 0),
                       axis=1, keepdims=True)
        pos_ref[pl.ds(c * _CH, _CH), :] = posc

    gidx = jax.lax.broadcasted_iota(jnp.int32, (1, GP), 1) * BLK
    be = jnp.zeros((1, GP), jnp.int32)
    for e in range(1, E):
        be = be + (gidx >= seg_sc[e]).astype(jnp.int32)
    blk_ref[...] = be


def _gating(x, Wg):
    return pl.pallas_call(
        _gate_body,
        out_shape=(
            jax.ShapeDtypeStruct((N, K), jnp.float32),
            jax.ShapeDtypeStruct((P, 1), jnp.int32),
            jax.ShapeDtypeStruct((1, GP), jnp.int32),
        ),
    )(x, Wg)


_SPW = P // NW


def _dispatch_body(x_hbm, pos_hbm, xs_hbm, idx_v, buf_v, sem):
    wid = lax.axis_index("s") * _NC + lax.axis_index("c")
    base = pl.multiple_of(wid * _SPW, _SPW)
    tokbase = pl.multiple_of(base & (N - 1), _SPW)
    pltpu.sync_copy(pos_hbm.at[pl.ds(base, _SPW)], idx_v)
    pltpu.sync_copy(x_hbm.at[pl.ds(tokbase, _SPW)], buf_v)
    pltpu.async_copy(buf_v, xs_hbm.at[idx_v], sem).wait()


def _dispatch(x, pos):
    mesh = plsc.VectorSubcoreMesh(core_axis_name="c", subcore_axis_name="s")
    return pl.kernel(
        _dispatch_body,
        mesh=mesh,
        out_type=jax.ShapeDtypeStruct((S, D), jnp.float32),
        scratch_types=[
            pltpu.VMEM((_SPW,), jnp.int32),
            pltpu.VMEM((_SPW, D), jnp.float32),
            pltpu.SemaphoreType.DMA,
        ],
        compiler_params=pltpu.CompilerParams(needs_layout_passes=False),
    )(x, pos)


def _mlp_body(be_ref, xs_ref, w1_ref, b1_ref, w2_ref, b2_ref, out_ref):
    xb = xs_ref[...].astype(jnp.bfloat16)
    h = jnp.maximum(
        jnp.dot(xb, w1_ref[0].astype(jnp.bfloat16),
                preferred_element_type=jnp.float32) + b1_ref[0], 0.0)
    o = jnp.dot(h.astype(jnp.bfloat16), w2_ref[0].astype(jnp.bfloat16),
                preferred_element_type=jnp.float32) + b2_ref[0]
    out_ref[...] = o


def _grouped_mlp(blk, xs, W1, b1, W2, b2):
    grid_spec = pltpu.PrefetchScalarGridSpec(
        num_scalar_prefetch=1,
        grid=(G,),
        in_specs=[
            pl.BlockSpec((BLK, D), lambda g, be: (g, 0)),
            pl.BlockSpec((1, D, H), lambda g, be: (be[g], 0, 0)),
            pl.BlockSpec((1, 1, H), lambda g, be: (be[g], 0, 0)),
            pl.BlockSpec((1, H, D), lambda g, be: (be[g], 0, 0)),
            pl.BlockSpec((1, 1, D), lambda g, be: (be[g], 0, 0)),
        ],
        out_specs=pl.BlockSpec((BLK, D), lambda g, be: (g, 0)),
    )
    return pl.pallas_call(
        _mlp_body,
        grid_spec=grid_spec,
        out_shape=jax.ShapeDtypeStruct((S, D), jnp.float32),
    )(blk, xs, W1, b1.reshape(E, 1, H), W2, b2.reshape(E, 1, D))


_TPW = N // NW


def _combine_body(out_hbm, pos_hbm, gf_hbm, y_hbm, idx0_v, idx1_v, g_v,
                  bufa_v, bufb_v, sem_a, sem_b):
    wid = lax.axis_index("s") * _NC + lax.axis_index("c")
    base = pl.multiple_of(wid * _TPW, _TPW)
    lane = lax.iota(jnp.int32, 16)
    pltpu.sync_copy(pos_hbm.at[pl.ds(base, _TPW)], idx0_v)
    pltpu.sync_copy(pos_hbm.at[pl.ds(N + base, _TPW)], idx1_v)
    pltpu.sync_copy(gf_hbm.at[pl.ds(2 * base, 2 * _TPW)], g_v)
    ca = pltpu.async_copy(out_hbm.at[idx0_v], bufa_v, sem_a)
    cb = pltpu.async_copy(out_hbm.at[idx1_v], bufb_v, sem_b)
    ca.wait()
    cb.wait()

    def grp(g8, _):
        gv = g_v[pl.ds(g8 * 16, 16)]
        for r8 in range(8):
            ga = jnp.sum(jnp.where(lane == 2 * r8, gv, 0.0))
            gb = jnp.sum(jnp.where(lane == 2 * r8 + 1, gv, 0.0))
            r = g8 * 8 + r8
            for k in range(D // 16):
                sl = pl.ds(k * 16, 16)
                bufa_v[r, sl] = bufa_v[r, sl] * ga + bufb_v[r, sl] * gb
        return _

    lax.fori_loop(0, _TPW // 8, grp, 0)
    pltpu.sync_copy(bufa_v, y_hbm.at[pl.ds(base, _TPW)])


def _combine(outs, pos, gf):
    mesh = plsc.VectorSubcoreMesh(core_axis_name="c", subcore_axis_name="s")
    return pl.kernel(
        _combine_body,
        mesh=mesh,
        out_type=jax.ShapeDtypeStruct((N, D), jnp.float32),
        scratch_types=[
            pltpu.VMEM((_TPW,), jnp.int32),
            pltpu.VMEM((_TPW,), jnp.int32),
            pltpu.VMEM((2 * _TPW,), jnp.float32),
            pltpu.VMEM((_TPW, D), jnp.float32),
            pltpu.VMEM((_TPW, D), jnp.float32),
            pltpu.SemaphoreType.DMA,
            pltpu.SemaphoreType.DMA,
        ],
        compiler_params=pltpu.CompilerParams(needs_layout_passes=False),
    )(outs, pos, gf)


def kernel(x, Wg, W1, b1, W2, b2):
    gp, pos2, blk2 = _gating(x, Wg)
    gf = gp.reshape(-1)
    pos = pos2.reshape(-1)
    blk = blk2.reshape(-1)
    xs = _dispatch(x, pos)
    outs = _grouped_mlp(blk, xs, W1, b1, W2, b2)
    return _combine(outs, pos, gf)


# speedup vs baseline: 2.1503x; 1.0481x over previous
import jax
import jax.numpy as jnp
from jax import lax
from jax.experimental import pallas as pl
from jax.experimental.pallas import tpu as pltpu
from jax.experimental.pallas import tpu_sc as plsc

N, D, H, E, K = 2048, 768, 768, 8, 2
P = N * K
BLK = 512
S = P + E * BLK
G = S // BLK
GP = 48
_CH = 256

_NC, _NS = 2, 16
NW = _NC * _NS


def _gate_body(x_ref, wg_ref, gp_ref, pos_ref, blk_ref):
    x = x_ref[...]
    logits = jnp.dot(x, wg_ref[...], preferred_element_type=jnp.float32)
    cols = jax.lax.broadcasted_iota(jnp.int32, logits.shape, 1)
    m1 = jnp.max(logits, axis=1, keepdims=True)
    a1 = jnp.argmax(logits, axis=1).reshape(-1, 1)
    neg = jnp.full_like(logits, -jnp.inf)
    masked = jnp.where(cols == a1, neg, logits)
    m2 = jnp.max(masked, axis=1, keepdims=True)
    a2 = jnp.argmax(masked, axis=1).reshape(-1, 1)
    t = jnp.exp(m2 - m1)
    w1g = 1.0 / (1.0 + t)
    w2g = t / (1.0 + t)
    gp_ref[...] = jnp.concatenate([w1g, w2g], axis=1)

    oh = jnp.concatenate([cols == a1, cols == a2], axis=0)
    ohf = oh.astype(jnp.float32)
    tot = jnp.sum(ohf, axis=0, keepdims=True).astype(jnp.int32)
    aligned = (tot + (BLK - 1)) & (-BLK)
    lane8 = jax.lax.broadcasted_iota(jnp.int32, (1, E), 1)
    seg_sc = []
    run = jnp.int32(0)
    for e in range(E):
        seg_sc.append(run)
        run = run + jnp.sum(jnp.where(lane8 == e, aligned, 0))
    seg_row = jnp.zeros((1, E), jnp.int32)
    for e in range(1, E):
        seg_row = seg_row + jnp.where(lane8 == e, seg_sc[e], 0)

    ri = jax.lax.broadcasted_iota(jnp.int32, (_CH, _CH), 0)
    ci = jax.lax.broadcasted_iota(jnp.int32, (_CH, _CH), 1)
    tril = (ri > ci).astype(jnp.float32)
    carry = jnp.zeros((1, E), jnp.float32)
    for c in range(P // _CH):
        ohc = ohf[c * _CH:(c + 1) * _CH]
        rk = jnp.dot(tril, ohc, preferred_element_type=jnp.float32) + carry
        carry = carry + jnp.sum(ohc, axis=0, keepdims=True)
        ohi = oh[c * _CH:(c + 1) * _CH]
        posc = jnp.sum(jnp.where(ohi, rk.astype(jnp.int32) + seg_row, 0),
                       axis=1, keepdims=True)
        pos_ref[pl.ds(c * _CH, _CH), :] = posc

    gidx = jax.lax.broadcasted_iota(jnp.int32, (1, GP), 1) * BLK
    be = jnp.zeros((1, GP), jnp.int32)
    for e in range(1, E):
        be = be + (gidx >= seg_sc[e]).astype(jnp.int32)
    blk_ref[...] = be


def _gating(x, Wg):
    return pl.pallas_call(
        _gate_body,
        out_shape=(
            jax.ShapeDtypeStruct((N, K), jnp.float32),
            jax.ShapeDtypeStruct((P, 1), jnp.int32),
            jax.ShapeDtypeStruct((1, GP), jnp.int32),
        ),
    )(x, Wg)


_SPW = P // NW


def _dispatch_body(x_hbm, pos_hbm, xs_hbm, idx_v, buf_v, sem):
    wid = lax.axis_index("s") * _NC + lax.axis_index("c")
    base = pl.multiple_of(wid * _SPW, _SPW)
    tokbase = pl.multiple_of(base & (N - 1), _SPW)
    pltpu.sync_copy(pos_hbm.at[pl.ds(base, _SPW)], idx_v)
    pltpu.sync_copy(x_hbm.at[pl.ds(tokbase, _SPW)], buf_v)
    pltpu.async_copy(buf_v, xs_hbm.at[idx_v], sem).wait()


def _dispatch(x, pos):
    mesh = plsc.VectorSubcoreMesh(core_axis_name="c", subcore_axis_name="s")
    return pl.kernel(
        _dispatch_body,
        mesh=mesh,
        out_type=jax.ShapeDtypeStruct((S, D), jnp.float32),
        scratch_types=[
            pltpu.VMEM((_SPW,), jnp.int32),
            pltpu.VMEM((_SPW, D), jnp.float32),
            pltpu.SemaphoreType.DMA,
        ],
        compiler_params=pltpu.CompilerParams(needs_layout_passes=False),
    )(x, pos)


def _mlp_body(be_ref, xs_ref, w1_ref, b1_ref, w2_ref, b2_ref, out_ref):
    xb = xs_ref[...].astype(jnp.bfloat16)
    h = jnp.maximum(
        jnp.dot(xb, w1_ref[0].astype(jnp.bfloat16),
                preferred_element_type=jnp.float32) + b1_ref[0], 0.0)
    o = jnp.dot(h.astype(jnp.bfloat16), w2_ref[0].astype(jnp.bfloat16),
                preferred_element_type=jnp.float32) + b2_ref[0]
    out_ref[...] = o


def _grouped_mlp(blk, xs, W1, b1, W2, b2):
    grid_spec = pltpu.PrefetchScalarGridSpec(
        num_scalar_prefetch=1,
        grid=(G,),
        in_specs=[
            pl.BlockSpec((BLK, D), lambda g, be: (g, 0)),
            pl.BlockSpec((1, D, H), lambda g, be: (be[g], 0, 0)),
            pl.BlockSpec((1, 1, H), lambda g, be: (be[g], 0, 0)),
            pl.BlockSpec((1, H, D), lambda g, be: (be[g], 0, 0)),
            pl.BlockSpec((1, 1, D), lambda g, be: (be[g], 0, 0)),
        ],
        out_specs=pl.BlockSpec((BLK, D), lambda g, be: (g, 0)),
    )
    return pl.pallas_call(
        _mlp_body,
        grid_spec=grid_spec,
        out_shape=jax.ShapeDtypeStruct((S, D), jnp.float32),
    )(blk, xs, W1, b1.reshape(E, 1, H), W2, b2.reshape(E, 1, D))


_TPW = N // NW


def _combine_body(out_hbm, pos_hbm, gf_hbm, y_hbm, idx0_v, idx1_v, g_v,
                  bufa_v, bufb_v, sem_a, sem_b):
    wid = lax.axis_index("s") * _NC + lax.axis_index("c")
    base = pl.multiple_of(wid * _TPW, _TPW)
    lane = lax.iota(jnp.int32, 16)
    pltpu.sync_copy(pos_hbm.at[pl.ds(base, _TPW)], idx0_v)
    pltpu.sync_copy(pos_hbm.at[pl.ds(N + base, _TPW)], idx1_v)
    pltpu.sync_copy(gf_hbm.at[pl.ds(2 * base, 2 * _TPW)], g_v)
    ca = pltpu.async_copy(out_hbm.at[idx0_v], bufa_v, sem_a)
    cb = pltpu.async_copy(out_hbm.at[idx1_v], bufb_v, sem_b)
    ca.wait()
    cb.wait()

    def grp(g8, _):
        gv = g_v[pl.ds(g8 * 16, 16)]
        for r8 in range(8):
            ga = jnp.sum(jnp.where(lane == 2 * r8, gv, 0.0))
            gb = jnp.sum(jnp.where(lane == 2 * r8 + 1, gv, 0.0))
            r = g8 * 8 + r8
            for k in range(D // 16):
                sl = pl.ds(k * 16, 16)
                bufa_v[r, sl] = bufa_v[r, sl] * ga + bufb_v[r, sl] * gb
        return _

    lax.fori_loop(0, _TPW // 8, grp, 0)
    pltpu.sync_copy(bufa_v, y_hbm.at[pl.ds(base, _TPW)])


def _combine(outs, pos, gf):
    mesh = plsc.VectorSubcoreMesh(core_axis_name="c", subcore_axis_name="s")
    return pl.kernel(
        _combine_body,
        mesh=mesh,
        out_type=jax.ShapeDtypeStruct((N, D), jnp.float32),
        scratch_types=[
            pltpu.VMEM((_TPW,), jnp.int32),
            pltpu.VMEM((_TPW,), jnp.int32),
            pltpu.VMEM((2 * _TPW,), jnp.float32),
            pltpu.VMEM((_TPW, D), jnp.float32),
            pltpu.VMEM((_TPW, D), jnp.float32),
            pltpu.SemaphoreType.DMA,
            pltpu.SemaphoreType.DMA,
        ],
        compiler_params=pltpu.CompilerParams(needs_layout_passes=False),
    )(outs, pos, gf)


def kernel(x, Wg, W1, b1, W2, b2):
    gp, pos2, blk2 = _gating(x, Wg)
    gf = gp.reshape(-1)
    pos = pos2.reshape(-1)
    blk = blk2.reshape(-1)
    xs = _dispatch(x, pos)
    outs = _grouped_mlp(blk, xs, W1, b1, W2, b2)
    return _combine(outs, pos, gf)
